# baseline (speedup 1.0000x reference)
import functools

import jax
import jax.numpy as jnp
from jax import lax
from jax.experimental import pallas as pl
from jax.experimental.pallas import tpu as pltpu
from jax.experimental.pallas import tpu_sc as plsc

B, A, NBH = 4, 2500, 32
F = 128
NS = 50
CUTOFF = 5.0
E = B * A * NBH
ROWS = B * A
TROWS = 14336
PADN = 4096

def _in2f_body(x_ref, w_ref, o_ref):
    o_ref[...] = jnp.dot(x_ref[...], w_ref[...],
                         preferred_element_type=jnp.float32)


def _in2f(x2, w):
    return pl.pallas_call(
        _in2f_body,
        grid=(14,),
        in_specs=[
            pl.BlockSpec((TROWS // 14, F), lambda i: (i, 0)),
            pl.BlockSpec((F, F), lambda i: (0, 0)),
        ],
        out_specs=pl.BlockSpec((TROWS // 14, F), lambda i: (i, 0)),
        out_shape=jax.ShapeDtypeStruct((TROWS, F), jnp.float32),
    )(x2, w)


NW = 32
PER_W = E // NW
CH = 80
N_CH = PER_W // CH


def _sc_gather(table, idx):
    mesh = plsc.VectorSubcoreMesh(core_axis_name="c", subcore_axis_name="s")

    @functools.partial(
        pl.kernel,
        mesh=mesh,
        compiler_params=pltpu.CompilerParams(use_tc_tiling_on_sc=True),
        out_type=jax.ShapeDtypeStruct((E, F), jnp.float32),
        scratch_types=[
            pltpu.VMEM((CH,), jnp.int32),
            pltpu.VMEM((CH, F), jnp.float32),
            pltpu.SemaphoreType.DMA,
        ],
    )
    def k(table_hbm, idx_hbm, out_hbm, idx_v, rows_v, sem):
        wid = lax.axis_index("s") * 2 + lax.axis_index("c")
        base = wid * PER_W

        def body(i, carry):
            start = base + i * CH
            pltpu.sync_copy(idx_hbm.at[pl.ds(start, CH)], idx_v)
            pltpu.async_copy(table_hbm.at[idx_v], rows_v, sem).wait()
            pltpu.sync_copy(rows_v, out_hbm.at[pl.ds(start, CH)])
            return carry

        lax.fori_loop(0, N_CH, body, 0)

    return k(table, idx)


TA = 200
EB = TA * NBH


def _main_body(g_ref, y_ref, w0_ref, b0_ref, w1_ref, b1_ref,
               w2_ref, b2_ref, w3_ref, b3_ref, w4_ref, b4_ref, o_ref):
    g = g_ref[...].reshape(EB, NS)
    h = jax.nn.gelu(jnp.dot(g, w0_ref[...],
                            preferred_element_type=jnp.float32) + b0_ref[...])
    h = jax.nn.gelu(jnp.dot(h, w1_ref[...],
                            preferred_element_type=jnp.float32) + b1_ref[...])
    wf = jnp.dot(h, w2_ref[...],
                 preferred_element_type=jnp.float32) + b2_ref[...]
    p = wf * y_ref[...]
    agg = p.reshape(TA, NBH, F).sum(axis=1)
    v = jax.nn.gelu(jnp.dot(agg, w3_ref[...],
                            preferred_element_type=jnp.float32) + b3_ref[...])
    o_ref[...] = jnp.dot(v, w4_ref[...],
                         preferred_element_type=jnp.float32) + b4_ref[...]


def _main(g2, y_nbh, w0, b0, w1, b1, w2, b2, w3, b3, w4, b4):
    n_blk = ROWS // TA
    full = lambda i: (0, 0)
    return pl.pallas_call(
        _main_body,
        grid=(n_blk,),
        in_specs=[
            pl.BlockSpec((TA, NBH, NS), lambda i: (i, 0, 0)),
            pl.BlockSpec((EB, F), lambda i: (i, 0)),
            pl.BlockSpec((NS, F), full),
            pl.BlockSpec((1, F), full),
            pl.BlockSpec((F, F), full),
            pl.BlockSpec((1, F), full),
            pl.BlockSpec((F, F), full),
            pl.BlockSpec((1, F), full),
            pl.BlockSpec((F, F), full),
            pl.BlockSpec((1, F), full),
            pl.BlockSpec((F, F), full),
            pl.BlockSpec((1, F), full),
        ],
        out_specs=pl.BlockSpec((TA, F), lambda i: (i, 0)),
        out_shape=jax.ShapeDtypeStruct((ROWS, F), jnp.float32),
    )(g2, y_nbh, w0, b0, w1, b1, w2, b2, w3, b3, w4, b4)


def kernel(x, r_ij, neighbors, neighbor_mask, f_ij,
           W_filt0, b_filt0, W_filt1, b_filt1, W_filt2, b_filt2,
           W_in2f, W_f2out, b_f2out, W_dense, b_dense):
    x_pad = jnp.concatenate(
        [x.reshape(ROWS, F), jnp.zeros((TROWS - ROWS, F), jnp.float32)])
    y = _in2f(x_pad, W_in2f)
    keep = (neighbor_mask != 0.0) & (r_ij < CUTOFF)
    flat = (neighbors.astype(jnp.int32)
            + (jnp.arange(B, dtype=jnp.int32) * A)[:, None, None]).reshape(E)
    pad_row = ROWS + (jnp.arange(E, dtype=jnp.int32) & (PADN - 1))
    flat_idx = jnp.where(keep.reshape(E), flat, pad_row)
    y_nbh = _sc_gather(y, flat_idx)
    out = _main(
        f_ij.reshape(ROWS, NBH, NS), y_nbh,
        W_filt0, b_filt0.reshape(1, F), W_filt1, b_filt1.reshape(1, F),
        W_filt2, b_filt2.reshape(1, F), W_f2out, b_f2out.reshape(1, F),
        W_dense, b_dense.reshape(1, F))
    return out.reshape(B, A, F)


# speedup vs baseline: 1.3019x; 1.3019x over previous
import functools

import jax
import jax.numpy as jnp
from jax import lax
from jax.experimental import pallas as pl
from jax.experimental.pallas import tpu as pltpu
from jax.experimental.pallas import tpu_sc as plsc

B, A, NBH = 4, 2500, 32
F = 128
NS = 50
CUTOFF = 5.0
TA = 128
A2 = 2560
NJ = A2 // TA
E2 = B * A2 * NBH
ROWS2 = B * A2
PADN = 4096
TROWS = ROWS2 + PADN
EBN = NBH * TA

def _in2f_body(x_ref, w_ref, o_ref):
    o_ref[...] = jnp.dot(x_ref[...], w_ref[...],
                         preferred_element_type=jnp.float32)


def _in2f(x2, w):
    return pl.pallas_call(
        _in2f_body,
        grid=(14,),
        in_specs=[
            pl.BlockSpec((TROWS // 14, F), lambda i: (i, 0)),
            pl.BlockSpec((F, F), lambda i: (0, 0)),
        ],
        out_specs=pl.BlockSpec((TROWS // 14, F), lambda i: (i, 0)),
        out_shape=jax.ShapeDtypeStruct((TROWS, F), jnp.float32),
    )(x2, w)


NW = 32
PER_W = E2 // NW
CH = 80
N_CH = PER_W // CH


def _sc_gather(table, idx):
    mesh = plsc.VectorSubcoreMesh(core_axis_name="c", subcore_axis_name="s")

    @functools.partial(
        pl.kernel,
        mesh=mesh,
        out_type=jax.ShapeDtypeStruct((E2, F), jnp.float32),
        scratch_types=[
            pltpu.VMEM((CH,), jnp.int32),
            pltpu.VMEM((CH, F), jnp.float32),
            pltpu.SemaphoreType.DMA,
        ],
    )
    def k(table_hbm, idx_hbm, out_hbm, idx_v, rows_v, sem):
        wid = lax.axis_index("s") * 2 + lax.axis_index("c")
        base = wid * PER_W

        def body(i, carry):
            start = base + i * CH
            pltpu.sync_copy(idx_hbm.at[pl.ds(start, CH)], idx_v)
            pltpu.async_copy(table_hbm.at[idx_v], rows_v, sem).wait()
            pltpu.sync_copy(rows_v, out_hbm.at[pl.ds(start, CH)])
            return carry

        lax.fori_loop(0, N_CH, body, 0)

    return k(table, idx)


def _main_body(g_ref, y_ref, w0_ref, b0_ref, w1_ref, b1_ref,
               w2_ref, b2_ref, w3_ref, b3_ref, w4_ref, b4_ref, o_ref):
    gT = g_ref[...].reshape(NS, EBN)
    h = jax.nn.gelu(jnp.dot(w0_ref[...], gT,
                            preferred_element_type=jnp.float32) + b0_ref[...])
    h = jax.nn.gelu(jnp.dot(w1_ref[...], h,
                            preferred_element_type=jnp.float32) + b1_ref[...])
    wfT = jnp.dot(w2_ref[...], h,
                  preferred_element_type=jnp.float32) + b2_ref[...]
    wf = jnp.transpose(wfT)
    p = wf * y_ref[...].reshape(EBN, F)
    agg = p.reshape(NBH, TA, F).sum(axis=0)
    v = jax.nn.gelu(jnp.dot(agg, w3_ref[...],
                            preferred_element_type=jnp.float32) + b3_ref[...])
    o_ref[...] = jnp.dot(v, w4_ref[...],
                         preferred_element_type=jnp.float32) + b4_ref[...]


def _main(ft, y3, w0t, b0c, w1t, b1c, w2t, b2c, w3, b3, w4, b4):
    full2 = lambda b, j: (0, 0)
    return pl.pallas_call(
        _main_body,
        grid=(B, NJ),
        in_specs=[
            pl.BlockSpec((1, NS, NBH, TA), lambda b, j: (b, 0, 0, j)),
            pl.BlockSpec((1, NBH, TA, F), lambda b, j: (b, 0, j, 0)),
            pl.BlockSpec((F, NS), full2),
            pl.BlockSpec((F, 1), full2),
            pl.BlockSpec((F, F), full2),
            pl.BlockSpec((F, 1), full2),
            pl.BlockSpec((F, F), full2),
            pl.BlockSpec((F, 1), full2),
            pl.BlockSpec((F, F), full2),
            pl.BlockSpec((1, F), full2),
            pl.BlockSpec((F, F), full2),
            pl.BlockSpec((1, F), full2),
        ],
        out_specs=pl.BlockSpec((TA, F), lambda b, j: (b * NJ + j, 0)),
        out_shape=jax.ShapeDtypeStruct((ROWS2, F), jnp.float32),
    )(ft, y3, w0t, b0c, w1t, b1c, w2t, b2c, w3, b3, w4, b4)


def kernel(x, r_ij, neighbors, neighbor_mask, f_ij,
           W_filt0, b_filt0, W_filt1, b_filt1, W_filt2, b_filt2,
           W_in2f, W_f2out, b_f2out, W_dense, b_dense):
    x_pad = jnp.concatenate(
        [jnp.pad(x, ((0, 0), (0, A2 - A), (0, 0))).reshape(ROWS2, F),
         jnp.zeros((PADN, F), jnp.float32)])
    y = _in2f(x_pad, W_in2f)

    nbr_t = jnp.pad(jnp.transpose(neighbors.astype(jnp.int32), (0, 2, 1)),
                    ((0, 0), (0, 0), (0, A2 - A)))
    r_t = jnp.pad(jnp.transpose(r_ij, (0, 2, 1)),
                  ((0, 0), (0, 0), (0, A2 - A)),
                  constant_values=2.0 * CUTOFF)
    m_t = jnp.pad(jnp.transpose(neighbor_mask, (0, 2, 1)),
                  ((0, 0), (0, 0), (0, A2 - A)))
    keep = (m_t != 0.0) & (r_t < CUTOFF)
    flat = nbr_t + (jnp.arange(B, dtype=jnp.int32) * A2)[:, None, None]
    spread = (ROWS2 + (jnp.arange(E2, dtype=jnp.int32) & (PADN - 1))
              ).reshape(B, NBH, A2)
    flat_idx = jnp.where(keep, flat, spread).reshape(E2)

    y_nbh = _sc_gather(y, flat_idx)

    ft = jnp.transpose(f_ij, (0, 3, 2, 1))
    y3 = y_nbh.reshape(B, NBH, A2, F)
    out = _main(
        ft, y3,
        W_filt0.T, b_filt0.reshape(F, 1),
        W_filt1.T, b_filt1.reshape(F, 1),
        W_filt2.T, b_filt2.reshape(F, 1),
        W_f2out, b_f2out.reshape(1, F),
        W_dense, b_dense.reshape(1, F))
    return out.reshape(B, A2, F)[:, :A, :]


# speedup vs baseline: 1.5543x; 1.1939x over previous
import functools

import jax
import jax.numpy as jnp
from jax import lax
from jax.experimental import pallas as pl
from jax.experimental.pallas import tpu as pltpu
from jax.experimental.pallas import tpu_sc as plsc

B, A, NBH = 4, 2500, 32
F = 128
NS = 50
CUTOFF = 5.0
TA = 128
A2 = 2560
NJ = A2 // TA
E2 = B * A2 * NBH
ROWS2 = B * A2
PADN = 4096
TROWS = ROWS2 + PADN
EBN = NBH * TA

def _in2f_body(x_ref, w_ref, o_ref):
    o_ref[...] = jnp.dot(x_ref[...], w_ref[...],
                         preferred_element_type=jnp.float32)


def _in2f(x2, w):
    return pl.pallas_call(
        _in2f_body,
        grid=(14,),
        in_specs=[
            pl.BlockSpec((TROWS // 14, F), lambda i: (i, 0)),
            pl.BlockSpec((F, F), lambda i: (0, 0)),
        ],
        out_specs=pl.BlockSpec((TROWS // 14, F), lambda i: (i, 0)),
        out_shape=jax.ShapeDtypeStruct((TROWS, F), jnp.float32),
    )(x2, w)


NW = 32
PER_W = E2 // NW
CH = 80
N_CH = PER_W // CH


def _sc_gather(table, idx):
    mesh = plsc.VectorSubcoreMesh(core_axis_name="c", subcore_axis_name="s")

    @functools.partial(
        pl.kernel,
        mesh=mesh,
        out_type=jax.ShapeDtypeStruct((E2, F), jnp.float32),
        scratch_types=[
            pltpu.VMEM((CH,), jnp.int32),
            pltpu.VMEM((CH, F), jnp.float32),
            pltpu.SemaphoreType.DMA,
        ],
    )
    def k(table_hbm, idx_hbm, out_hbm, idx_v, rows_v, sem):
        wid = lax.axis_index("s") * 2 + lax.axis_index("c")
        base = wid * PER_W

        def body(i, carry):
            start = base + i * CH
            pltpu.sync_copy(idx_hbm.at[pl.ds(start, CH)], idx_v)
            pltpu.async_copy(table_hbm.at[idx_v], rows_v, sem).wait()
            pltpu.sync_copy(rows_v, out_hbm.at[pl.ds(start, CH)])
            return carry

        lax.fori_loop(0, N_CH, body, 0)

    return k(table, idx)


def _filter_body(g_ref, w0_ref, b0_ref, w1_ref, b1_ref, w2_ref, b2_ref,
                 o_ref):
    gT = g_ref[...].reshape(NS, EBN)
    h = jax.nn.gelu(jnp.dot(w0_ref[...], gT,
                            preferred_element_type=jnp.float32) + b0_ref[...])
    h = jax.nn.gelu(jnp.dot(w1_ref[...], h,
                            preferred_element_type=jnp.float32) + b1_ref[...])
    wfT = jnp.dot(w2_ref[...], h,
                  preferred_element_type=jnp.float32) + b2_ref[...]
    o_ref[...] = jnp.transpose(wfT)


def _filter(ft, w0t, b0c, w1t, b1c, w2t, b2c):
    full2 = lambda b, j: (0, 0)
    return pl.pallas_call(
        _filter_body,
        grid=(B, NJ),
        in_specs=[
            pl.BlockSpec((1, NS, NBH, TA), lambda b, j: (b, 0, 0, j)),
            pl.BlockSpec((F, NS), full2),
            pl.BlockSpec((F, 1), full2),
            pl.BlockSpec((F, F), full2),
            pl.BlockSpec((F, 1), full2),
            pl.BlockSpec((F, F), full2),
            pl.BlockSpec((F, 1), full2),
        ],
        out_specs=pl.BlockSpec((EBN, F), lambda b, j: (b * NJ + j, 0)),
        out_shape=jax.ShapeDtypeStruct((E2, F), jnp.float32),
    )(ft, w0t, b0c, w1t, b1c, w2t, b2c)


def _combine_body(wf_ref, y_ref, w3_ref, b3_ref, w4_ref, b4_ref, o_ref):
    p = wf_ref[...] * y_ref[...].reshape(EBN, F)
    agg = p.reshape(NBH, TA, F).sum(axis=0)
    v = jax.nn.gelu(jnp.dot(agg, w3_ref[...],
                            preferred_element_type=jnp.float32) + b3_ref[...])
    o_ref[...] = jnp.dot(v, w4_ref[...],
                         preferred_element_type=jnp.float32) + b4_ref[...]


def _combine(wf, y3, w3, b3, w4, b4):
    full2 = lambda b, j: (0, 0)
    return pl.pallas_call(
        _combine_body,
        grid=(B, NJ),
        in_specs=[
            pl.BlockSpec((EBN, F), lambda b, j: (b * NJ + j, 0)),
            pl.BlockSpec((1, NBH, TA, F), lambda b, j: (b, 0, j, 0)),
            pl.BlockSpec((F, F), full2),
            pl.BlockSpec((1, F), full2),
            pl.BlockSpec((F, F), full2),
            pl.BlockSpec((1, F), full2),
        ],
        out_specs=pl.BlockSpec((TA, F), lambda b, j: (b * NJ + j, 0)),
        out_shape=jax.ShapeDtypeStruct((ROWS2, F), jnp.float32),
    )(wf, y3, w3, b3, w4, b4)


def kernel(x, r_ij, neighbors, neighbor_mask, f_ij,
           W_filt0, b_filt0, W_filt1, b_filt1, W_filt2, b_filt2,
           W_in2f, W_f2out, b_f2out, W_dense, b_dense):
    x_pad = jnp.concatenate(
        [jnp.pad(x, ((0, 0), (0, A2 - A), (0, 0))).reshape(ROWS2, F),
         jnp.zeros((PADN, F), jnp.float32)])
    y = _in2f(x_pad, W_in2f)

    nbr_t = jnp.pad(jnp.transpose(neighbors.astype(jnp.int32), (0, 2, 1)),
                    ((0, 0), (0, 0), (0, A2 - A)))
    r_t = jnp.pad(jnp.transpose(r_ij, (0, 2, 1)),
                  ((0, 0), (0, 0), (0, A2 - A)),
                  constant_values=2.0 * CUTOFF)
    m_t = jnp.pad(jnp.transpose(neighbor_mask, (0, 2, 1)),
                  ((0, 0), (0, 0), (0, A2 - A)))
    keep = (m_t != 0.0) & (r_t < CUTOFF)
    flat = nbr_t + (jnp.arange(B, dtype=jnp.int32) * A2)[:, None, None]
    spread = (ROWS2 + (jnp.arange(E2, dtype=jnp.int32) & (PADN - 1))
              ).reshape(B, NBH, A2)
    flat_idx = jnp.where(keep, flat, spread).reshape(E2)

    y_nbh = _sc_gather(y, flat_idx)

    ft = jnp.transpose(f_ij, (0, 3, 2, 1))
    wf = _filter(
        ft,
        W_filt0.T, b_filt0.reshape(F, 1),
        W_filt1.T, b_filt1.reshape(F, 1),
        W_filt2.T, b_filt2.reshape(F, 1))
    y3 = y_nbh.reshape(B, NBH, A2, F)
    out = _combine(wf, y3, W_f2out, b_f2out.reshape(1, F),
                   W_dense, b_dense.reshape(1, F))
    return out.reshape(B, A2, F)[:, :A, :]
